# Initial kernel scaffold; baseline (speedup 1.0000x reference)
import jax
import jax.numpy as jnp
from jax.experimental import pallas as pl


def kernel(x, edge_index, edge_f, edge_attr, W, b):
    raise NotImplementedError("write your pallas kernel here")


# speedup vs baseline: 7.9234x; 7.9234x over previous
import functools

import jax
import jax.numpy as jnp
from jax import lax
from jax.experimental import pallas as pl
from jax.experimental.pallas import tpu as pltpu
from jax.experimental.pallas import tpu_sc as plsc

NC = 2
NS = 16
LANES = 16
NW = NC * NS

SENTINEL = -3.0e38
THRESH = -1.0e37


def _tc1_body(x_ref, a_ref, wb_ref, b_ref, pbt_ref, qt_ref):
    x_blk = x_ref[...]
    d = x_blk.shape[1]
    i5 = jnp.eye(d, dtype=jnp.float32)
    dn = (((1,), (1,)), ((), ()))
    xt = lax.dot_general(i5, x_blk, dn, preferred_element_type=jnp.float32,
                          precision=lax.Precision.HIGHEST)
    dn0 = (((0,), (0,)), ((), ()))
    pbt = lax.dot_general(a_ref[...], xt, dn0,
                          preferred_element_type=jnp.float32,
                          precision=lax.Precision.HIGHEST)
    pbt_ref[...] = pbt + b_ref[...].reshape(d, 1)
    qt_ref[...] = lax.dot_general(wb_ref[...], xt, dn0,
                                  preferred_element_type=jnp.float32,
                          precision=lax.Precision.HIGHEST)


def _sc_body(n, e, d, ch, qt_hbm, src_hbm, dst_hbm, part_hbm,
             qcol, acc, sbuf, dbuf):
    epw = e // NW
    cid = lax.axis_index("c")
    sid = lax.axis_index("s")
    w = sid * NC + cid
    e0 = w * epw
    zeros16 = jnp.zeros((LANES,), jnp.float32)

    for col in range(d):
        pltpu.sync_copy(qt_hbm.at[pl.ds(col * n, n)], qcol)

        def init_body(i, carry):
            acc[pl.ds(i * LANES, LANES)] = zeros16 + SENTINEL
            return carry
        lax.fori_loop(0, n // LANES, init_body, 0)

        def chunk_body(k, carry):
            off = e0 + k * ch
            pltpu.sync_copy(src_hbm.at[pl.ds(off, ch)], sbuf)
            pltpu.sync_copy(dst_hbm.at[pl.ds(off, ch)], dbuf)

            def grp_body(i, c2):
                dv = dbuf[pl.ds(i * LANES, LANES)]
                sv = sbuf[pl.ds(i * LANES, LANES)]
                q = plsc.load_gather(qcol, [sv])
                a = plsc.load_gather(acc, [dv])
                plsc.store_scatter(acc, [dv], q, mask=q > a)
                a2 = plsc.load_gather(acc, [dv])
                viol = q > a2

                @pl.when(jnp.any(viol))
                def _fixup():
                    def rnd(_, c3):
                        a3 = plsc.load_gather(acc, [dv])
                        plsc.store_scatter(acc, [dv], q, mask=q > a3)
                        return c3
                    lax.fori_loop(0, LANES - 1, rnd, 0)
                return c2
            lax.fori_loop(0, ch // LANES, grp_body, 0)
            return carry
        lax.fori_loop(0, epw // ch, chunk_body, 0)

        pltpu.sync_copy(acc, part_hbm.at[pl.ds((w * d + col) * n, n)])


def _tc2_body(part_ref, pbt_ref, out_ref):
    m = jnp.max(part_ref[...], axis=0)
    d = m.shape[0]
    valt = jnp.where(m > THRESH, m + pbt_ref[...], 0.0)
    i5 = jnp.eye(d, dtype=jnp.float32)
    dn0 = (((0,), (0,)), ((), ()))
    out_ref[...] = lax.dot_general(valt, i5, dn0,
                                   preferred_element_type=jnp.float32,
                          precision=lax.Precision.HIGHEST)


def kernel(x, edge_index, edge_f, edge_attr, W, b):
    del edge_f, edge_attr
    n, d = x.shape
    e = edge_index.shape[1]
    npad = ((n + 127) // 128) * 128
    assert npad % LANES == 0 and e % NW == 0
    epw = e // NW
    ch = 4000 if epw % 4000 == 0 else epw
    assert ch % LANES == 0 and epw % ch == 0
    units = npad // 128
    u = max(v for v in range(1, min(units, 32) + 1) if units % v == 0)
    nb = u * 128

    edge_index = edge_index.astype(jnp.int32)
    src = edge_index[0]
    dst = edge_index[1]
    a_mat = W[:d] - W[d:]
    wb_mat = W[d:]
    xp = jnp.pad(x, ((0, npad - n), (0, 0)))

    pbt, qt = pl.pallas_call(
        _tc1_body,
        grid=(npad // nb,),
        in_specs=[
            pl.BlockSpec((nb, d), lambda i: (i, 0)),
            pl.BlockSpec((d, d), lambda i: (0, 0)),
            pl.BlockSpec((d, d), lambda i: (0, 0)),
            pl.BlockSpec((d,), lambda i: (0,)),
        ],
        out_specs=[
            pl.BlockSpec((d, nb), lambda i: (0, i)),
            pl.BlockSpec((d, nb), lambda i: (0, i)),
        ],
        out_shape=[
            jax.ShapeDtypeStruct((d, npad), jnp.float32),
            jax.ShapeDtypeStruct((d, npad), jnp.float32),
        ],
    )(xp, a_mat, wb_mat, b)

    mesh = plsc.VectorSubcoreMesh(
        core_axis_name="c", subcore_axis_name="s",
        num_cores=NC, num_subcores=NS)
    partial = pl.kernel(
        functools.partial(_sc_body, npad, e, d, ch),
        out_type=jax.ShapeDtypeStruct((NW * d * npad,), jnp.float32),
        mesh=mesh,
        compiler_params=pltpu.CompilerParams(needs_layout_passes=False),
        scratch_types=[
            pltpu.VMEM((npad,), jnp.float32),
            pltpu.VMEM((npad,), jnp.float32),
            pltpu.VMEM((ch,), jnp.int32),
            pltpu.VMEM((ch,), jnp.int32),
        ],
    )(qt.reshape(-1), src, dst)
    partial = partial.reshape(NW, d, npad)

    out = pl.pallas_call(
        _tc2_body,
        grid=(npad // nb,),
        in_specs=[
            pl.BlockSpec((NW, d, nb), lambda i: (0, 0, i)),
            pl.BlockSpec((d, nb), lambda i: (0, i)),
        ],
        out_specs=pl.BlockSpec((nb, d), lambda i: (i, 0)),
        out_shape=jax.ShapeDtypeStruct((npad, d), jnp.float32),
    )(partial, pbt)
    return out[:n]


# speedup vs baseline: 35.6337x; 4.4973x over previous
import functools

import jax
import jax.numpy as jnp
from jax import lax
from jax.experimental import pallas as pl
from jax.experimental.pallas import tpu as pltpu
from jax.experimental.pallas import tpu_sc as plsc

NC = 2
NS = 16
LANES = 16
NW = NC * NS

SENTINEL = -3.0e38
THRESH = -1.0e37


def _tc1_body(x_ref, a_ref, wb_ref, b_ref, pbt_ref, qt_ref):
    x_blk = x_ref[...]
    d = x_blk.shape[1]
    i5 = jnp.eye(d, dtype=jnp.float32)
    dn = (((1,), (1,)), ((), ()))
    xt = lax.dot_general(i5, x_blk, dn, preferred_element_type=jnp.float32,
                          precision=lax.Precision.HIGHEST)
    dn0 = (((0,), (0,)), ((), ()))
    pbt = lax.dot_general(a_ref[...], xt, dn0,
                          preferred_element_type=jnp.float32,
                          precision=lax.Precision.HIGHEST)
    pbt_ref[...] = pbt + b_ref[...].reshape(d, 1)
    qt_ref[...] = lax.dot_general(wb_ref[...], xt, dn0,
                                  preferred_element_type=jnp.float32,
                          precision=lax.Precision.HIGHEST)


UNROLL = 5


def _sc_body(n, e, d, ch, qt_hbm, src_hbm, dst_hbm, sent_hbm, part_hbm,
             qcol, acc, sbuf, dbuf):
    epw = e // NW
    cid = lax.axis_index("c")
    sid = lax.axis_index("s")
    w = sid * NC + cid
    e0 = w * epw
    nu = ch // (LANES * UNROLL)

    for col in range(d):
        pltpu.sync_copy(qt_hbm.at[pl.ds(col * n, n)], qcol)
        pltpu.sync_copy(sent_hbm, acc)

        def chunk_body(k, carry):
            off = e0 + k * ch
            pltpu.sync_copy(src_hbm.at[pl.ds(off, ch)], sbuf)
            pltpu.sync_copy(dst_hbm.at[pl.ds(off, ch)], dbuf)

            def blk_body(i, c2):
                base = i * (LANES * UNROLL)
                dvs, qs = [], []
                for j in range(UNROLL):
                    dv = dbuf[pl.ds(base + j * LANES, LANES)]
                    sv = sbuf[pl.ds(base + j * LANES, LANES)]
                    dvs.append(dv)
                    qs.append(plsc.load_gather(qcol, [sv]))
                avs = [plsc.load_gather(acc, [dv]) for dv in dvs]
                for j in range(UNROLL):
                    plsc.store_scatter(acc, [dvs[j]], qs[j],
                                       mask=qs[j] > avs[j])
                viol = None
                for j in range(UNROLL):
                    a2 = plsc.load_gather(acc, [dvs[j]])
                    v = qs[j] > a2
                    viol = v if viol is None else (viol | v)

                @pl.when(jnp.any(viol))
                def _fixup():
                    for j in range(UNROLL):
                        a3 = plsc.load_gather(acc, [dvs[j]])
                        plsc.store_scatter(acc, [dvs[j]], qs[j],
                                           mask=qs[j] > a3)
                    v2 = None
                    for j in range(UNROLL):
                        a4 = plsc.load_gather(acc, [dvs[j]])
                        vj = qs[j] > a4
                        v2 = vj if v2 is None else (v2 | vj)

                    @pl.when(jnp.any(v2))
                    def _fixup_dups():
                        def rnd(_, c3):
                            for j in range(UNROLL):
                                a5 = plsc.load_gather(acc, [dvs[j]])
                                plsc.store_scatter(acc, [dvs[j]], qs[j],
                                                   mask=qs[j] > a5)
                            return c3
                        lax.fori_loop(0, LANES - 1, rnd, 0)
                return c2
            lax.fori_loop(0, nu, blk_body, 0)
            return carry
        lax.fori_loop(0, epw // ch, chunk_body, 0)

        pltpu.sync_copy(acc, part_hbm.at[pl.ds((w * d + col) * n, n)])


def _tc2_body(part_ref, pbt_ref, out_ref):
    m = jnp.max(part_ref[...], axis=0)
    d = m.shape[0]
    valt = jnp.where(m > THRESH, m + pbt_ref[...], 0.0)
    i5 = jnp.eye(d, dtype=jnp.float32)
    dn0 = (((0,), (0,)), ((), ()))
    out_ref[...] = lax.dot_general(valt, i5, dn0,
                                   preferred_element_type=jnp.float32,
                          precision=lax.Precision.HIGHEST)


def kernel(x, edge_index, edge_f, edge_attr, W, b):
    del edge_f, edge_attr
    n, d = x.shape
    e = edge_index.shape[1]
    npad = ((n + 127) // 128) * 128
    assert npad % LANES == 0 and e % NW == 0
    epw = e // NW
    ch = 10000 if epw % 10000 == 0 else epw
    assert ch % (LANES * 5) == 0 and epw % ch == 0
    units = npad // 128
    u = max(v for v in range(1, min(units, 32) + 1) if units % v == 0)
    nb = u * 128

    edge_index = edge_index.astype(jnp.int32)
    src = edge_index[0]
    dst = edge_index[1]
    a_mat = W[:d] - W[d:]
    wb_mat = W[d:]
    xp = jnp.pad(x, ((0, npad - n), (0, 0)))

    pbt, qt = pl.pallas_call(
        _tc1_body,
        grid=(npad // nb,),
        in_specs=[
            pl.BlockSpec((nb, d), lambda i: (i, 0)),
            pl.BlockSpec((d, d), lambda i: (0, 0)),
            pl.BlockSpec((d, d), lambda i: (0, 0)),
            pl.BlockSpec((d,), lambda i: (0,)),
        ],
        out_specs=[
            pl.BlockSpec((d, nb), lambda i: (0, i)),
            pl.BlockSpec((d, nb), lambda i: (0, i)),
        ],
        out_shape=[
            jax.ShapeDtypeStruct((d, npad), jnp.float32),
            jax.ShapeDtypeStruct((d, npad), jnp.float32),
        ],
    )(xp, a_mat, wb_mat, b)

    mesh = plsc.VectorSubcoreMesh(
        core_axis_name="c", subcore_axis_name="s",
        num_cores=NC, num_subcores=NS)
    partial = pl.kernel(
        functools.partial(_sc_body, npad, e, d, ch),
        out_type=jax.ShapeDtypeStruct((NW * d * npad,), jnp.float32),
        mesh=mesh,
        compiler_params=pltpu.CompilerParams(needs_layout_passes=False),
        scratch_types=[
            pltpu.VMEM((npad,), jnp.float32),
            pltpu.VMEM((npad,), jnp.float32),
            pltpu.VMEM((ch,), jnp.int32),
            pltpu.VMEM((ch,), jnp.int32),
        ],
    )(qt.reshape(-1), src, dst, jnp.full((npad,), SENTINEL, jnp.float32))
    partial = partial.reshape(NW, d, npad)

    out = pl.pallas_call(
        _tc2_body,
        grid=(npad // nb,),
        in_specs=[
            pl.BlockSpec((NW, d, nb), lambda i: (0, 0, i)),
            pl.BlockSpec((d, nb), lambda i: (0, i)),
        ],
        out_specs=pl.BlockSpec((nb, d), lambda i: (i, 0)),
        out_shape=jax.ShapeDtypeStruct((npad, d), jnp.float32),
    )(partial, pbt)
    return out[:n]


# speedup vs baseline: 76.8521x; 2.1567x over previous
import functools

import jax
import jax.numpy as jnp
from jax import lax
from jax.experimental import pallas as pl
from jax.experimental.pallas import tpu as pltpu
from jax.experimental.pallas import tpu_sc as plsc

NC = 2
NS = 16
LANES = 16
NW = NC * NS
UNROLL = 5

SENTINEL = -3.0e38
THRESH = -1.0e37


def _tc1_body(x_ref, a_ref, wb_ref, b_ref, pbt_ref, qt_ref):
    x_blk = x_ref[...]
    d = x_blk.shape[1]
    i5 = jnp.eye(d, dtype=jnp.float32)
    dn1 = (((1,), (1,)), ((), ()))
    dn0 = (((0,), (0,)), ((), ()))
    xt = lax.dot_general(i5, x_blk, dn1, preferred_element_type=jnp.float32,
                         precision=lax.Precision.HIGHEST)
    pbt = lax.dot_general(a_ref[...], xt, dn0,
                          preferred_element_type=jnp.float32,
                          precision=lax.Precision.HIGHEST)
    pbt_ref[...] = pbt + b_ref[...].reshape(d, 1)
    qt_ref[...] = lax.dot_general(wb_ref[...], xt, dn0,
                                  preferred_element_type=jnp.float32,
                                  precision=lax.Precision.HIGHEST)


def _pack_body(ei_ref, pk_ref):
    ei = ei_ref[...]
    s = ei[0].astype(jnp.uint32)
    dd = ei[1].astype(jnp.uint32)
    pk_ref[...] = (dd << 16) | s


def _extract(pk):
    sv = (pk & jnp.uint32(0xFFFF)).astype(jnp.int32)
    dv = (pk >> 16).astype(jnp.int32)
    return sv, dv


def _sc_body(n, e, d, ch, qt_hbm, pk_hbm, part_hbm,
             qcol, acc, pbuf0, pbuf1, sem0, sem1):
    epw = e // NW
    nch = epw // ch
    cid = lax.axis_index("c")
    sid = lax.axis_index("s")
    w = sid * NC + cid
    e0 = w * epw
    nu = ch // (LANES * UNROLL)
    sent16 = jnp.full((LANES,), SENTINEL, jnp.float32)
    zero16 = jnp.zeros((LANES,), jnp.int32)
    one16 = jnp.ones((LANES,), jnp.int32)
    bufs = ((pbuf0, sem0), (pbuf1, sem1))

    def chunk_src(c):
        return pk_hbm.at[pl.ds(e0 + c * ch, ch)]

    for col in range(d):
        pltpu.sync_copy(qt_hbm.at[pl.ds(col * n, n)], qcol)

        def init_body(i, carry):
            for j in range(8):
                acc[pl.ds((i * 8 + j) * LANES, LANES)] = sent16
            return carry
        lax.fori_loop(0, n // (LANES * 8), init_body, 0)

        pltpu.async_copy(chunk_src(0), pbuf0, sem0)

        def pair_body(k, carry):
            for bsel in range(2):
                pbuf, sem = bufs[bsel]
                nxt, nsem = bufs[1 - bsel]
                c = k * 2 + bsel
                pltpu.make_async_copy(chunk_src(c), pbuf, sem).wait()

                @pl.when(c < nch - 1)
                def _prefetch():
                    pltpu.async_copy(chunk_src(c + 1), nxt, nsem)

                def blk_body(i, vacc):
                    base = i * (LANES * UNROLL)
                    svs, dvs, qs = [], [], []
                    for j in range(UNROLL):
                        pk = pbuf[pl.ds(base + j * LANES, LANES)]
                        sv, dv = _extract(pk)
                        svs.append(sv)
                        dvs.append(dv)
                        qs.append(plsc.load_gather(qcol, [sv]))
                    avs = [plsc.load_gather(acc, [dv]) for dv in dvs]
                    for j in range(UNROLL):
                        plsc.store_scatter(acc, [dvs[j]], qs[j],
                                           mask=qs[j] > avs[j])
                    a1s = [plsc.load_gather(acc, [dv]) for dv in dvs]
                    for j in range(UNROLL):
                        plsc.store_scatter(acc, [dvs[j]], qs[j],
                                           mask=qs[j] > a1s[j])
                    viol = None
                    for j in range(UNROLL):
                        a2 = plsc.load_gather(acc, [dvs[j]])
                        v = qs[j] > a2
                        viol = v if viol is None else (viol | v)
                    return vacc | jnp.where(viol, one16, zero16)

                vacc = lax.fori_loop(0, nu, blk_body, zero16)

                @pl.when(jnp.max(vacc) > 0)
                def _deep_repair():
                    def grp(g, c2):
                        pk = pbuf[pl.ds(g * LANES, LANES)]
                        sv, dv = _extract(pk)
                        q = plsc.load_gather(qcol, [sv])

                        def rnd(r, c3):
                            a = plsc.load_gather(acc, [dv])
                            plsc.store_scatter(acc, [dv], q, mask=q > a)
                            return c3
                        lax.fori_loop(0, LANES, rnd, 0)
                        return c2
                    lax.fori_loop(0, ch // LANES, grp, 0)
            return carry
        lax.fori_loop(0, nch // 2, pair_body, 0)

        pltpu.sync_copy(acc, part_hbm.at[pl.ds((w * d + col) * n, n)])


def _tc2_body(part_ref, pbt_ref, out_ref):
    m = jnp.max(part_ref[...], axis=0)
    d = m.shape[0]
    valt = jnp.where(m > THRESH, m + pbt_ref[...], 0.0)
    i5 = jnp.eye(d, dtype=jnp.float32)
    dn0 = (((0,), (0,)), ((), ()))
    out_ref[...] = lax.dot_general(valt, i5, dn0,
                                   preferred_element_type=jnp.float32,
                                   precision=lax.Precision.HIGHEST)


def kernel(x, edge_index, edge_f, edge_attr, W, b):
    del edge_f, edge_attr
    n, d = x.shape
    e = edge_index.shape[1]
    npad = ((n + 127) // 128) * 128
    assert n < (1 << 16) and npad % (LANES * 8) == 0 and e % NW == 0
    epw = e // NW
    ch = 10000 if epw % 10000 == 0 else epw
    assert ch % (LANES * UNROLL) == 0 and epw % ch == 0
    assert (epw // ch) % 2 == 0
    units = npad // 128
    u = max(v for v in range(1, min(units, 32) + 1) if units % v == 0)
    nb = u * 128
    eu = e // 128
    ue = max(v for v in range(1, min(eu, 1024) + 1) if eu % v == 0)
    eb = ue * 128

    edge_index = edge_index.astype(jnp.int32)
    a_mat = W[:d] - W[d:]
    wb_mat = W[d:]
    xp = jnp.pad(x, ((0, npad - n), (0, 0)))

    pbt, qt = pl.pallas_call(
        _tc1_body,
        grid=(npad // nb,),
        in_specs=[
            pl.BlockSpec((nb, d), lambda i: (i, 0)),
            pl.BlockSpec((d, d), lambda i: (0, 0)),
            pl.BlockSpec((d, d), lambda i: (0, 0)),
            pl.BlockSpec((d,), lambda i: (0,)),
        ],
        out_specs=[
            pl.BlockSpec((d, nb), lambda i: (0, i)),
            pl.BlockSpec((d, nb), lambda i: (0, i)),
        ],
        out_shape=[
            jax.ShapeDtypeStruct((d, npad), jnp.float32),
            jax.ShapeDtypeStruct((d, npad), jnp.float32),
        ],
    )(xp, a_mat, wb_mat, b)

    packed = pl.pallas_call(
        _pack_body,
        grid=(e // eb,),
        in_specs=[pl.BlockSpec((2, eb), lambda i: (0, i))],
        out_specs=pl.BlockSpec((eb,), lambda i: (i,)),
        out_shape=jax.ShapeDtypeStruct((e,), jnp.uint32),
    )(edge_index)

    mesh = plsc.VectorSubcoreMesh(
        core_axis_name="c", subcore_axis_name="s",
        num_cores=NC, num_subcores=NS)
    partial = pl.kernel(
        functools.partial(_sc_body, npad, e, d, ch),
        out_type=jax.ShapeDtypeStruct((NW * d * npad,), jnp.float32),
        mesh=mesh,
        compiler_params=pltpu.CompilerParams(needs_layout_passes=False),
        scratch_types=[
            pltpu.VMEM((npad,), jnp.float32),
            pltpu.VMEM((npad,), jnp.float32),
            pltpu.VMEM((ch,), jnp.uint32),
            pltpu.VMEM((ch,), jnp.uint32),
            pltpu.SemaphoreType.DMA,
            pltpu.SemaphoreType.DMA,
        ],
    )(qt.reshape(-1), packed)
    partial = partial.reshape(NW, d, npad)

    out = pl.pallas_call(
        _tc2_body,
        grid=(npad // nb,),
        in_specs=[
            pl.BlockSpec((NW, d, nb), lambda i: (0, 0, i)),
            pl.BlockSpec((d, nb), lambda i: (0, i)),
        ],
        out_specs=pl.BlockSpec((nb, d), lambda i: (i, 0)),
        out_shape=jax.ShapeDtypeStruct((npad, d), jnp.float32),
    )(partial, pbt)
    return out[:n]


# speedup vs baseline: 94.4713x; 1.2293x over previous
import functools

import jax
import jax.numpy as jnp
from jax import lax
from jax.experimental import pallas as pl
from jax.experimental.pallas import tpu as pltpu
from jax.experimental.pallas import tpu_sc as plsc

NC = 2
NS = 16
LANES = 16
NW = NC * NS
UNROLL = 5

SENTINEL = -3.0e38
THRESH = -1.0e37


def _tc1_body(xt_ref, a_ref, wb_ref, b_ref, pbt_ref, qt_ref):
    xt = xt_ref[...]
    d = xt.shape[0]
    dn0 = (((0,), (0,)), ((), ()))
    pbt = lax.dot_general(a_ref[...], xt, dn0,
                          preferred_element_type=jnp.float32,
                          precision=lax.Precision.HIGHEST)
    pbt_ref[...] = pbt + b_ref[...].reshape(d, 1)
    qt_ref[...] = lax.dot_general(wb_ref[...], xt, dn0,
                                  preferred_element_type=jnp.float32,
                                  precision=lax.Precision.HIGHEST)


def _pack_body(ei_ref, pk_ref):
    ei = ei_ref[...]
    s = ei[0].astype(jnp.uint32)
    dd = ei[1].astype(jnp.uint32)
    pk_ref[...] = (dd << 16) | s


def _extract(pk):
    sv = (pk & jnp.uint32(0xFFFF)).astype(jnp.int32)
    dv = (pk >> 16).astype(jnp.int32)
    return sv, dv


def _sc_body(n, e, d, ch, qt_hbm, pk_hbm, part_hbm,
             qcol, acc, pbuf0, pbuf1, sem0, sem1):
    epw = e // NW
    nch = epw // ch
    cid = lax.axis_index("c")
    sid = lax.axis_index("s")
    w = sid * NC + cid
    e0 = w * epw
    nu = ch // (LANES * UNROLL)
    sent16 = jnp.full((LANES,), SENTINEL, jnp.float32)
    zero16 = jnp.zeros((LANES,), jnp.int32)
    one16 = jnp.ones((LANES,), jnp.int32)
    bufs = ((pbuf0, sem0), (pbuf1, sem1))

    def chunk_src(c):
        return pk_hbm.at[pl.ds(e0 + c * ch, ch)]

    for col in range(d):
        pltpu.sync_copy(qt_hbm.at[pl.ds(col * n, n)], qcol)

        def init_body(i, carry):
            for j in range(8):
                acc[pl.ds((i * 8 + j) * LANES, LANES)] = sent16
            return carry
        lax.fori_loop(0, n // (LANES * 8), init_body, 0)

        pltpu.async_copy(chunk_src(0), pbuf0, sem0)

        def pair_body(k, carry):
            for bsel in range(2):
                pbuf, sem = bufs[bsel]
                nxt, nsem = bufs[1 - bsel]
                c = k * 2 + bsel
                pltpu.make_async_copy(chunk_src(c), pbuf, sem).wait()

                @pl.when(c < nch - 1)
                def _prefetch():
                    pltpu.async_copy(chunk_src(c + 1), nxt, nsem)

                def blk_body(i, vacc):
                    base = i * (LANES * UNROLL)
                    svs, dvs, qs = [], [], []
                    for j in range(UNROLL):
                        pk = pbuf[pl.ds(base + j * LANES, LANES)]
                        sv, dv = _extract(pk)
                        svs.append(sv)
                        dvs.append(dv)
                        qs.append(plsc.load_gather(qcol, [sv]))
                    avs = [plsc.load_gather(acc, [dv]) for dv in dvs]
                    for j in range(UNROLL):
                        plsc.store_scatter(acc, [dvs[j]], qs[j],
                                           mask=qs[j] > avs[j])
                    a1s = [plsc.load_gather(acc, [dv]) for dv in dvs]
                    for j in range(UNROLL):
                        plsc.store_scatter(acc, [dvs[j]], qs[j],
                                           mask=qs[j] > a1s[j])
                    viol = None
                    for j in range(UNROLL):
                        a2 = plsc.load_gather(acc, [dvs[j]])
                        v = qs[j] > a2
                        viol = v if viol is None else (viol | v)
                    return vacc | jnp.where(viol, one16, zero16)

                vacc = lax.fori_loop(0, nu, blk_body, zero16)

                @pl.when(jnp.max(vacc) > 0)
                def _deep_repair():
                    def grp(g, c2):
                        pk = pbuf[pl.ds(g * LANES, LANES)]
                        sv, dv = _extract(pk)
                        q = plsc.load_gather(qcol, [sv])

                        def rnd(r, c3):
                            a = plsc.load_gather(acc, [dv])
                            plsc.store_scatter(acc, [dv], q, mask=q > a)
                            return c3
                        lax.fori_loop(0, LANES, rnd, 0)
                        return c2
                    lax.fori_loop(0, ch // LANES, grp, 0)
            return carry
        lax.fori_loop(0, nch // 2, pair_body, 0)

        pltpu.sync_copy(acc, part_hbm.at[pl.ds((w * d + col) * n, n)])


def _tc2_body(part_ref, pbt_ref, out_ref):
    m = jnp.max(part_ref[...], axis=0)
    out_ref[...] = jnp.where(m > THRESH, m + pbt_ref[...], 0.0)


def kernel(x, edge_index, edge_f, edge_attr, W, b):
    del edge_f, edge_attr
    n, d = x.shape
    e = edge_index.shape[1]
    npad = ((n + 127) // 128) * 128
    assert n < (1 << 16) and npad % (LANES * 8) == 0 and e % NW == 0
    epw = e // NW
    ch = 10000 if epw % 10000 == 0 else epw
    assert ch % (LANES * UNROLL) == 0 and epw % ch == 0
    assert (epw // ch) % 2 == 0
    units = npad // 128
    u = max(v for v in range(1, min(units, 32) + 1) if units % v == 0)
    nb = u * 128
    eu = e // 128
    ue = max(v for v in range(1, min(eu, 1024) + 1) if eu % v == 0)
    eb = ue * 128

    edge_index = edge_index.astype(jnp.int32)
    a_mat = W[:d] - W[d:]
    wb_mat = W[d:]
    xtp = jnp.pad(x.T, ((0, 0), (0, npad - n)))

    pbt, qt = pl.pallas_call(
        _tc1_body,
        grid=(npad // nb,),
        in_specs=[
            pl.BlockSpec((d, nb), lambda i: (0, i)),
            pl.BlockSpec((d, d), lambda i: (0, 0)),
            pl.BlockSpec((d, d), lambda i: (0, 0)),
            pl.BlockSpec((d,), lambda i: (0,)),
        ],
        out_specs=[
            pl.BlockSpec((d, nb), lambda i: (0, i)),
            pl.BlockSpec((d, nb), lambda i: (0, i)),
        ],
        out_shape=[
            jax.ShapeDtypeStruct((d, npad), jnp.float32),
            jax.ShapeDtypeStruct((d, npad), jnp.float32),
        ],
    )(xtp, a_mat, wb_mat, b)

    packed = pl.pallas_call(
        _pack_body,
        grid=(e // eb,),
        in_specs=[pl.BlockSpec((2, eb), lambda i: (0, i))],
        out_specs=pl.BlockSpec((eb,), lambda i: (i,)),
        out_shape=jax.ShapeDtypeStruct((e,), jnp.uint32),
    )(edge_index)

    mesh = plsc.VectorSubcoreMesh(
        core_axis_name="c", subcore_axis_name="s",
        num_cores=NC, num_subcores=NS)
    partial = pl.kernel(
        functools.partial(_sc_body, npad, e, d, ch),
        out_type=jax.ShapeDtypeStruct((NW * d * npad,), jnp.float32),
        mesh=mesh,
        compiler_params=pltpu.CompilerParams(needs_layout_passes=False),
        scratch_types=[
            pltpu.VMEM((npad,), jnp.float32),
            pltpu.VMEM((npad,), jnp.float32),
            pltpu.VMEM((ch,), jnp.uint32),
            pltpu.VMEM((ch,), jnp.uint32),
            pltpu.SemaphoreType.DMA,
            pltpu.SemaphoreType.DMA,
        ],
    )(qt.reshape(-1), packed)
    partial = partial.reshape(NW, d, npad)

    outt = pl.pallas_call(
        _tc2_body,
        grid=(npad // nb,),
        in_specs=[
            pl.BlockSpec((NW, d, nb), lambda i: (0, 0, i)),
            pl.BlockSpec((d, nb), lambda i: (0, i)),
        ],
        out_specs=pl.BlockSpec((d, nb), lambda i: (0, i)),
        out_shape=jax.ShapeDtypeStruct((d, npad), jnp.float32),
    )(partial, pbt)
    return outt[:, :n].T
